# baseline (speedup 1.0000x reference)
import functools

import jax
import jax.numpy as jnp
from jax import lax
from jax.experimental import pallas as pl
from jax.experimental.pallas import tpu as pltpu
from jax.experimental.pallas import tpu_sc as plsc

N = 10000
D = 128
E = 320000
NC = 2
NS = 16
NW = NC * NS
CHUNK = 128
CPW = 80
NBUF = 2
GRP = 16
E_PAD = NW * CPW * CHUNK
N_ACC = 10240
RPT = N_ACC // NS
EPS = 1e-5

_MESH = plsc.VectorSubcoreMesh(core_axis_name="c", subcore_axis_name="s")


def _zero_fill(buf):
    @pl.loop(0, CHUNK)
    def _(r):
        @pl.loop(0, D // 16)
        def _(cc):
            buf[r, pl.ds(cc * 16, 16)] = jnp.zeros((16,), jnp.float32)


def _sc_aggregate(h, src3, dst3):
    @functools.partial(
        pl.kernel,
        out_type=jax.ShapeDtypeStruct((NC, N_ACC, D), jnp.float32),
        mesh=_MESH,
        scratch_types=[
            pltpu.VMEM((GRP, CHUNK), jnp.int32),
            pltpu.VMEM((CPW, CHUNK), jnp.int32),
            pltpu.VMEM((CHUNK, D), jnp.float32),
            pltpu.VMEM_SHARED((N_ACC, D), jnp.float32),
            pltpu.SemaphoreType.DMA,
        ],
    )
    def k(h_hbm, src_hbm, dst_hbm, agg_hbm, src_g, dst_v, gbuf, acc, sem):
        c = lax.axis_index("c")
        s = lax.axis_index("s")
        wid = c * NS + s

        pltpu.sync_copy(dst_hbm.at[wid], dst_v)

        _zero_fill(gbuf)
        for kk in range(RPT // CHUNK):
            pltpu.sync_copy(gbuf, acc.at[pl.ds(s * RPT + kk * CHUNK, CHUNK)])
        plsc.subcore_barrier()

        @pl.loop(0, CPW // GRP)
        def _(g):
            pltpu.sync_copy(src_hbm.at[wid, pl.ds(g * GRP, GRP)], src_g)

            @pl.loop(0, GRP)
            def _(kk):
                pltpu.async_copy(h_hbm.at[src_g.at[kk]], gbuf, sem).wait()
                pltpu.sync_copy(gbuf, acc.at[dst_v.at[g * GRP + kk]], add=True)

        plsc.subcore_barrier()
        pltpu.sync_copy(acc.at[pl.ds(s * RPT, RPT)],
                        agg_hbm.at[c, pl.ds(s * RPT, RPT)])

    return k(h, src3, dst3)


def _sc_degree(dst3):
    @functools.partial(
        pl.kernel,
        out_type=jax.ShapeDtypeStruct((NC, N_ACC, D), jnp.float32),
        mesh=_MESH,
        scratch_types=[
            pltpu.VMEM((CPW, CHUNK), jnp.int32),
            pltpu.VMEM((CHUNK, D), jnp.float32),
            pltpu.VMEM_SHARED((N_ACC, D), jnp.float32),
            pltpu.SemaphoreType.DMA,
        ],
    )
    def k(dst_hbm, deg_hbm, dst_v, obuf, dacc, sem):
        c = lax.axis_index("c")
        s = lax.axis_index("s")
        wid = c * NS + s

        pltpu.sync_copy(dst_hbm.at[wid], dst_v)

        _zero_fill(obuf)
        for kk in range(RPT // CHUNK):
            pltpu.sync_copy(obuf, dacc.at[pl.ds(s * RPT + kk * CHUNK, CHUNK)])

        @pl.loop(0, CHUNK)
        def _(r):
            obuf[r, pl.ds(0, 16)] = jnp.ones((16,), jnp.float32)

        plsc.subcore_barrier()

        @pl.loop(0, CPW)
        def _(j):
            pltpu.sync_copy(obuf, dacc.at[dst_v.at[j]], add=True)

        plsc.subcore_barrier()
        pltpu.sync_copy(dacc.at[pl.ds(s * RPT, RPT)],
                        deg_hbm.at[c, pl.ds(s * RPT, RPT)])

    return k(dst3)


BM = 1000
NB = N // BM


def _tc_matmul_stats(agg2, deg2, h, Wl, bl, Wr, with_stats):
    def body(agg_ref, deg_ref, h_ref, Wl_ref, bl_ref, Wr_ref, t_ref, *rest):
        i = pl.program_id(0)
        agg = agg_ref[0] + agg_ref[1]
        deg = deg_ref[0, :, 0:1] + deg_ref[1, :, 0:1]
        inv = 1.0 / jnp.maximum(deg, 1.0)
        t = (
            jnp.dot(agg * inv, Wl_ref[...], preferred_element_type=jnp.float32,
                    precision=lax.Precision.HIGHEST)
            + jnp.dot(h_ref[...], Wr_ref[...], preferred_element_type=jnp.float32,
                      precision=lax.Precision.HIGHEST)
            + bl_ref[...]
        )
        t_ref[...] = t
        if with_stats:
            (stats_ref,) = rest
            s1 = jnp.sum(t, axis=0, keepdims=True)
            s2 = jnp.sum(t * t, axis=0, keepdims=True)
            sval = jnp.concatenate(
                [s1, s2, jnp.zeros((6, D), jnp.float32)], axis=0)

            @pl.when(i == 0)
            def _():
                stats_ref[...] = sval

            @pl.when(i > 0)
            def _():
                stats_ref[...] += sval

    out_shape = [jax.ShapeDtypeStruct((N, D), jnp.float32)]
    out_specs = [pl.BlockSpec((BM, D), lambda i: (i, 0))]
    if with_stats:
        out_shape.append(jax.ShapeDtypeStruct((8, D), jnp.float32))
        out_specs.append(pl.BlockSpec((8, D), lambda i: (0, 0)))
    return pl.pallas_call(
        body,
        grid=(NB,),
        in_specs=[
            pl.BlockSpec((2, BM, D), lambda i: (0, i, 0)),
            pl.BlockSpec((2, BM, 8), lambda i: (0, i, 0)),
            pl.BlockSpec((BM, D), lambda i: (i, 0)),
            pl.BlockSpec((D, D), lambda i: (0, 0)),
            pl.BlockSpec((1, D), lambda i: (0, 0)),
            pl.BlockSpec((D, D), lambda i: (0, 0)),
        ],
        out_specs=out_specs if with_stats else out_specs[0],
        out_shape=out_shape if with_stats else out_shape[0],
    )(agg2, deg2, h, Wl, bl.reshape(1, D), Wr)


def _tc_bn_relu(t, stats, g, b):
    def body(t_ref, stats_ref, g_ref, b_ref, o_ref):
        mu = stats_ref[0:1, :] * (1.0 / N)
        ex2 = stats_ref[1:2, :] * (1.0 / N)
        var = ex2 - mu * mu
        o_ref[...] = jnp.maximum(
            g_ref[...] * (t_ref[...] - mu) * lax.rsqrt(var + EPS) + b_ref[...],
            0.0)

    return pl.pallas_call(
        body,
        grid=(NB,),
        in_specs=[
            pl.BlockSpec((BM, D), lambda i: (i, 0)),
            pl.BlockSpec((8, D), lambda i: (0, 0)),
            pl.BlockSpec((1, D), lambda i: (0, 0)),
            pl.BlockSpec((1, D), lambda i: (0, 0)),
        ],
        out_specs=pl.BlockSpec((BM, D), lambda i: (i, 0)),
        out_shape=jax.ShapeDtypeStruct((N, D), jnp.float32),
    )(t, stats, g.reshape(1, D), b.reshape(1, D))


def _tc_layer(agg2, deg2, h, Wl, bl, Wr, g, b, bn_relu):
    if bn_relu:
        t, stats = _tc_matmul_stats(agg2, deg2, h, Wl, bl, Wr, True)
        return _tc_bn_relu(t, stats, g, b)
    return _tc_matmul_stats(agg2, deg2, h, Wl, bl, Wr, False)


def kernel(x, edge_index, Wl0, bl0, Wr0, g0, b0, Wl1, bl1, Wr1, g1, b1,
           Wl2, bl2, Wr2):
    src = edge_index[0]
    dst = edge_index[1]
    pad = E_PAD - E
    pad_dst = (N + jnp.arange(pad, dtype=jnp.int32) % (N_ACC - N))
    src3 = jnp.concatenate(
        [src, jnp.zeros((pad,), jnp.int32)]).reshape(NW, CPW, CHUNK)
    dst3 = jnp.concatenate([dst, pad_dst]).reshape(NW, CPW, CHUNK)

    deg_full = _sc_degree(dst3)
    deg2 = deg_full[:, :, :8]

    agg2 = _sc_aggregate(x, src3, dst3)
    h1 = _tc_layer(agg2, deg2, x, Wl0, bl0, Wr0, g0, b0, True)
    agg2 = _sc_aggregate(h1, src3, dst3)
    h2 = _tc_layer(agg2, deg2, h1, Wl1, bl1, Wr1, g1, b1, True)
    agg2 = _sc_aggregate(h2, src3, dst3)
    return _tc_layer(agg2, deg2, h2, Wl2, bl2, Wr2, None, None, False)


# speedup vs baseline: 1.4540x; 1.4540x over previous
import functools

import jax
import jax.numpy as jnp
from jax import lax
from jax.experimental import pallas as pl
from jax.experimental.pallas import tpu as pltpu
from jax.experimental.pallas import tpu_sc as plsc

N = 10000
D = 128
E = 320000
NC = 2
NS = 16
NW = NC * NS
CHUNK = 128
CPW = 79
E_PAD = NW * CPW * CHUNK
N_ACC = 10240
RPT = N_ACC // NS
EPS = 1e-5

_MESH = plsc.VectorSubcoreMesh(core_axis_name="c", subcore_axis_name="s")


def _zero_fill(buf):
    @pl.loop(0, CHUNK)
    def _(r):
        @pl.loop(0, D // 16)
        def _(cc):
            buf[r, pl.ds(cc * 16, 16)] = jnp.zeros((16,), jnp.float32)


def _sc_aggregate(h, src3, dst3):
    @functools.partial(
        pl.kernel,
        out_type=jax.ShapeDtypeStruct((NC, N_ACC, D), jnp.float32),
        mesh=_MESH,
        scratch_types=[
            pltpu.VMEM((CPW, CHUNK), jnp.int32),
            pltpu.VMEM((CPW, CHUNK), jnp.int32),
            pltpu.VMEM((CHUNK, D), jnp.float32),
            pltpu.VMEM_SHARED((N_ACC, D), jnp.float32),
            pltpu.SemaphoreType.DMA,
        ],
    )
    def k(h_hbm, src_hbm, dst_hbm, agg_hbm, src_v, dst_v, gbuf, acc, sem):
        c = lax.axis_index("c")
        s = lax.axis_index("s")
        wid = c * NS + s

        pltpu.sync_copy(src_hbm.at[wid], src_v)
        pltpu.sync_copy(dst_hbm.at[wid], dst_v)

        _zero_fill(gbuf)
        for kk in range(RPT // CHUNK):
            pltpu.sync_copy(gbuf, acc.at[pl.ds(s * RPT + kk * CHUNK, CHUNK)])
        plsc.subcore_barrier()

        @pl.loop(0, CPW)
        def _(j):
            pltpu.async_copy(h_hbm.at[src_v.at[j]], gbuf, sem).wait()
            pltpu.sync_copy(gbuf, acc.at[dst_v.at[j]], add=True)

        plsc.subcore_barrier()
        pltpu.sync_copy(acc.at[pl.ds(s * RPT, RPT)],
                        agg_hbm.at[c, pl.ds(s * RPT, RPT)])

    return k(h, src3, dst3)


def _sc_degree(dst3):
    @functools.partial(
        pl.kernel,
        out_type=jax.ShapeDtypeStruct((NC, N_ACC, D), jnp.float32),
        mesh=_MESH,
        scratch_types=[
            pltpu.VMEM((CPW, CHUNK), jnp.int32),
            pltpu.VMEM((CHUNK, D), jnp.float32),
            pltpu.VMEM_SHARED((N_ACC, D), jnp.float32),
            pltpu.SemaphoreType.DMA,
        ],
    )
    def k(dst_hbm, deg_hbm, dst_v, obuf, dacc, sem):
        c = lax.axis_index("c")
        s = lax.axis_index("s")
        wid = c * NS + s

        pltpu.sync_copy(dst_hbm.at[wid], dst_v)

        _zero_fill(obuf)
        for kk in range(RPT // CHUNK):
            pltpu.sync_copy(obuf, dacc.at[pl.ds(s * RPT + kk * CHUNK, CHUNK)])

        @pl.loop(0, CHUNK)
        def _(r):
            obuf[r, pl.ds(0, 16)] = jnp.ones((16,), jnp.float32)

        plsc.subcore_barrier()

        @pl.loop(0, CPW)
        def _(j):
            pltpu.sync_copy(obuf, dacc.at[dst_v.at[j]], add=True)

        plsc.subcore_barrier()
        pltpu.sync_copy(dacc.at[pl.ds(s * RPT, RPT)],
                        deg_hbm.at[c, pl.ds(s * RPT, RPT)])

    return k(dst3)


BM = 1000
NB = N // BM


def _tc_matmul_stats(agg2, deg2, h, Wl, bl, Wr, with_stats):
    def body(agg_ref, deg_ref, h_ref, Wl_ref, bl_ref, Wr_ref, t_ref, *rest):
        i = pl.program_id(0)
        agg = agg_ref[0] + agg_ref[1]
        deg = deg_ref[0, :, 0:1] + deg_ref[1, :, 0:1]
        inv = 1.0 / jnp.maximum(deg, 1.0)
        t = (
            jnp.dot(agg * inv, Wl_ref[...], preferred_element_type=jnp.float32,
                    precision=lax.Precision.HIGHEST)
            + jnp.dot(h_ref[...], Wr_ref[...], preferred_element_type=jnp.float32,
                      precision=lax.Precision.HIGHEST)
            + bl_ref[...]
        )
        t_ref[...] = t
        if with_stats:
            (stats_ref,) = rest
            s1 = jnp.sum(t, axis=0, keepdims=True)
            s2 = jnp.sum(t * t, axis=0, keepdims=True)
            sval = jnp.concatenate(
                [s1, s2, jnp.zeros((6, D), jnp.float32)], axis=0)

            @pl.when(i == 0)
            def _():
                stats_ref[...] = sval

            @pl.when(i > 0)
            def _():
                stats_ref[...] += sval

    out_shape = [jax.ShapeDtypeStruct((N, D), jnp.float32)]
    out_specs = [pl.BlockSpec((BM, D), lambda i: (i, 0))]
    if with_stats:
        out_shape.append(jax.ShapeDtypeStruct((8, D), jnp.float32))
        out_specs.append(pl.BlockSpec((8, D), lambda i: (0, 0)))
    return pl.pallas_call(
        body,
        grid=(NB,),
        in_specs=[
            pl.BlockSpec((2, BM, D), lambda i: (0, i, 0)),
            pl.BlockSpec((2, BM, 8), lambda i: (0, i, 0)),
            pl.BlockSpec((BM, D), lambda i: (i, 0)),
            pl.BlockSpec((D, D), lambda i: (0, 0)),
            pl.BlockSpec((1, D), lambda i: (0, 0)),
            pl.BlockSpec((D, D), lambda i: (0, 0)),
        ],
        out_specs=out_specs if with_stats else out_specs[0],
        out_shape=out_shape if with_stats else out_shape[0],
    )(agg2, deg2, h, Wl, bl.reshape(1, D), Wr)


def _tc_bn_relu(t, stats, g, b):
    def body(t_ref, stats_ref, g_ref, b_ref, o_ref):
        mu = stats_ref[0:1, :] * (1.0 / N)
        ex2 = stats_ref[1:2, :] * (1.0 / N)
        var = ex2 - mu * mu
        o_ref[...] = jnp.maximum(
            g_ref[...] * (t_ref[...] - mu) * lax.rsqrt(var + EPS) + b_ref[...],
            0.0)

    return pl.pallas_call(
        body,
        grid=(NB,),
        in_specs=[
            pl.BlockSpec((BM, D), lambda i: (i, 0)),
            pl.BlockSpec((8, D), lambda i: (0, 0)),
            pl.BlockSpec((1, D), lambda i: (0, 0)),
            pl.BlockSpec((1, D), lambda i: (0, 0)),
        ],
        out_specs=pl.BlockSpec((BM, D), lambda i: (i, 0)),
        out_shape=jax.ShapeDtypeStruct((N, D), jnp.float32),
    )(t, stats, g.reshape(1, D), b.reshape(1, D))


def _tc_layer(agg2, deg2, h, Wl, bl, Wr, g, b, bn_relu):
    if bn_relu:
        t, stats = _tc_matmul_stats(agg2, deg2, h, Wl, bl, Wr, True)
        return _tc_bn_relu(t, stats, g, b)
    return _tc_matmul_stats(agg2, deg2, h, Wl, bl, Wr, False)


def kernel(x, edge_index, Wl0, bl0, Wr0, g0, b0, Wl1, bl1, Wr1, g1, b1,
           Wl2, bl2, Wr2):
    src = edge_index[0]
    dst = edge_index[1]
    pad = E_PAD - E
    pad_dst = (N + jnp.arange(pad, dtype=jnp.int32) % (N_ACC - N))
    src3 = jnp.concatenate(
        [src, jnp.zeros((pad,), jnp.int32)]).reshape(NW, CPW, CHUNK)
    dst3 = jnp.concatenate([dst, pad_dst]).reshape(NW, CPW, CHUNK)

    deg_full = _sc_degree(dst3)
    deg2 = deg_full[:, :, :8]

    agg2 = _sc_aggregate(x, src3, dst3)
    h1 = _tc_layer(agg2, deg2, x, Wl0, bl0, Wr0, g0, b0, True)
    agg2 = _sc_aggregate(h1, src3, dst3)
    h2 = _tc_layer(agg2, deg2, h1, Wl1, bl1, Wr1, g1, b1, True)
    agg2 = _sc_aggregate(h2, src3, dst3)
    return _tc_layer(agg2, deg2, h2, Wl2, bl2, Wr2, None, None, False)
